# speedup vs baseline: 1.5572x; 1.5572x over previous
import functools

import numpy as np
import jax
import jax.numpy as jnp
from jax import lax
from jax.experimental import pallas as pl
from jax.experimental.pallas import tpu as pltpu
from jax.experimental.pallas import tpu_sc as plsc

D_M = 128
BATCH = 1024
MAXLEN = 200
N_TOK = BATCH * MAXLEN
NC, NS, LANES = 2, 16, 16
NW = NC * NS
PER_W = N_TOK // NW
CHUNK = 40
NCHUNK = PER_W // CHUNK
PHASES = MAXLEN // CHUNK


def _pe_table() -> jnp.ndarray:
    pos = np.arange(MAXLEN, dtype=np.float64)[:, None]
    i = np.arange(D_M)[None, :]
    angle = pos / np.power(10000.0, (2.0 * (i // 2)) / float(D_M))
    pe = np.where(i % 2 == 0, np.sin(angle), np.cos(angle))
    return jnp.asarray(pe, dtype=jnp.float32)


_mesh = plsc.VectorSubcoreMesh(core_axis_name="c", subcore_axis_name="s")


@functools.partial(
    pl.kernel,
    out_type=jax.ShapeDtypeStruct((N_TOK, D_M), jnp.float32),
    mesh=_mesh,
    scratch_types=[
        pltpu.VMEM((PER_W,), jnp.int32),
        pltpu.VMEM((MAXLEN, D_M), jnp.float32),
        pltpu.VMEM((CHUNK, D_M), jnp.float32),
        pltpu.SemaphoreType.DMA,
    ],
)
def _emb_kernel(idx_hbm, pe_hbm, table_hbm, out_hbm, idx_v, pe_v, rows_v, sem):
    wid = lax.axis_index("s") * NC + lax.axis_index("c")
    base = wid * PER_W
    pltpu.sync_copy(pe_hbm, pe_v)
    pltpu.sync_copy(idx_hbm.at[pl.ds(base, PER_W)], idx_v)

    def chunk_body(t, carry):
        off = t * CHUNK
        pltpu.async_copy(
            table_hbm.at[idx_v.at[pl.ds(off, CHUNK)]], rows_v, sem
        ).wait()
        prow = lax.rem(t, PHASES) * CHUNK

        def add_row(r, c2):
            for j in range(D_M // LANES):
                sl = pl.ds(j * LANES, LANES)
                rows_v[r, sl] = rows_v[r, sl] + pe_v[prow + r, sl]
            return c2

        lax.fori_loop(0, CHUNK, add_row, 0)
        pltpu.sync_copy(rows_v, out_hbm.at[pl.ds(base + off, CHUNK)])
        return carry

    lax.fori_loop(0, NCHUNK, chunk_body, 0)


def kernel(inp, emb_table):
    idx = inp.reshape(N_TOK).astype(jnp.int32)
    out = _emb_kernel(idx, _pe_table(), emb_table)
    return out.reshape(BATCH, MAXLEN, D_M)


# speedup vs baseline: 2.5956x; 1.6668x over previous
import functools

import numpy as np
import jax
import jax.numpy as jnp
from jax import lax
from jax.experimental import pallas as pl
from jax.experimental.pallas import tpu as pltpu
from jax.experimental.pallas import tpu_sc as plsc

D_M = 128
BATCH = 1024
MAXLEN = 200
N_TOK = BATCH * MAXLEN
NC, NS, LANES = 2, 16, 16
NW = NC * NS
PER_W = N_TOK // NW
CHUNK = 40
NCHUNK = PER_W // CHUNK
PHASES = MAXLEN // CHUNK


def _pe_table() -> jnp.ndarray:
    pos = np.arange(MAXLEN, dtype=np.float64)[:, None]
    i = np.arange(D_M)[None, :]
    angle = pos / np.power(10000.0, (2.0 * (i // 2)) / float(D_M))
    pe = np.where(i % 2 == 0, np.sin(angle), np.cos(angle))
    return jnp.asarray(pe, dtype=jnp.float32)


NBUF = 4
ROW_UNROLL = 4

_mesh = plsc.VectorSubcoreMesh(core_axis_name="c", subcore_axis_name="s")


@functools.partial(
    pl.kernel,
    out_type=jax.ShapeDtypeStruct((N_TOK, D_M), jnp.float32),
    mesh=_mesh,
    scratch_types=[
        pltpu.VMEM((PER_W,), jnp.int32),
        pltpu.VMEM((MAXLEN, D_M), jnp.float32),
        pltpu.VMEM((NBUF, CHUNK, D_M), jnp.float32),
        pltpu.SemaphoreType.DMA,
        pltpu.SemaphoreType.DMA,
        pltpu.SemaphoreType.DMA,
        pltpu.SemaphoreType.DMA,
        pltpu.SemaphoreType.DMA,
        pltpu.SemaphoreType.DMA,
        pltpu.SemaphoreType.DMA,
        pltpu.SemaphoreType.DMA,
    ],
)
def _emb_kernel(idx_hbm, pe_hbm, table_hbm, out_hbm, idx_v, pe_v, rows_v,
                sg0, sg1, sg2, sg3, so0, so1, so2, so3):
    sem_g = (sg0, sg1, sg2, sg3)
    sem_o = (so0, so1, so2, so3)
    wid = lax.axis_index("s") * NC + lax.axis_index("c")
    base = wid * PER_W
    pltpu.sync_copy(pe_hbm, pe_v)
    pltpu.sync_copy(idx_hbm.at[pl.ds(base, PER_W)], idx_v)

    def start_gather(t, b):
        pltpu.async_copy(
            table_hbm.at[idx_v.at[pl.ds(t * CHUNK, CHUNK)]],
            rows_v.at[b], sem_g[b])

    def wait_gather(b):
        pltpu.make_async_copy(
            table_hbm.at[idx_v.at[pl.ds(0, CHUNK)]],
            rows_v.at[b], sem_g[b]).wait()

    def start_out(t, b):
        pltpu.async_copy(
            rows_v.at[b], out_hbm.at[pl.ds(base + t * CHUNK, CHUNK)],
            sem_o[b])

    def wait_out(b):
        pltpu.make_async_copy(
            rows_v.at[b], out_hbm.at[pl.ds(base, CHUNK)], sem_o[b]).wait()

    for b in range(NBUF):
        start_gather(b, b)

    def group_body(g, carry):
        for b in range(NBUF):
            u = g * NBUF + b
            b2 = (b + 1) % NBUF
            cond = (g >= 1) if b < NBUF - 1 else (g <= NCHUNK // NBUF - 2)

            @pl.when(cond)
            def _():
                wait_out(b2)
                start_gather(u + 1, b2)

            wait_gather(b)
            prow = lax.rem(u, PHASES) * CHUNK

            def add_rows(r0, c2):
                for k in range(ROW_UNROLL):
                    r = r0 * ROW_UNROLL + k
                    for j in range(D_M // LANES):
                        sl = pl.ds(j * LANES, LANES)
                        rows_v[b, r, sl] = rows_v[b, r, sl] + pe_v[prow + r, sl]
                return c2

            lax.fori_loop(0, CHUNK // ROW_UNROLL, add_rows, 0)
            start_out(u, b)
        return carry

    lax.fori_loop(0, NCHUNK // NBUF, group_body, 0)
    for b in range(NBUF):
        wait_out(b)


def kernel(inp, emb_table):
    idx = inp.reshape(N_TOK).astype(jnp.int32)
    out = _emb_kernel(idx, _pe_table(), emb_table)
    return out.reshape(BATCH, MAXLEN, D_M)
